# baseline (speedup 1.0000x reference)
import jax
import jax.numpy as jnp
from jax.experimental import pallas as pl
from jax.experimental.pallas import tpu as pltpu


def _copy_body(u_in, i_in, u_out, i_out, sem_u, sem_i):
    cu = pltpu.make_async_copy(u_in, u_out, sem_u)
    ci = pltpu.make_async_copy(i_in, i_out, sem_i)
    cu.start()
    ci.start()
    cu.wait()
    ci.wait()


def kernel(emb_user, emb_item):
    u, i = pl.pallas_call(
        _copy_body,
        in_specs=[
            pl.BlockSpec(memory_space=pl.ANY),
            pl.BlockSpec(memory_space=pl.ANY),
        ],
        out_specs=[
            pl.BlockSpec(memory_space=pl.ANY),
            pl.BlockSpec(memory_space=pl.ANY),
        ],
        out_shape=[
            jax.ShapeDtypeStruct(emb_user.shape, emb_user.dtype),
            jax.ShapeDtypeStruct(emb_item.shape, emb_item.dtype),
        ],
        scratch_shapes=[pltpu.SemaphoreType.DMA, pltpu.SemaphoreType.DMA],
    )(emb_user, emb_item)
    return (u, i)


# speedup vs baseline: 3.3153x; 3.3153x over previous
import jax
import jax.numpy as jnp
from jax.experimental import pallas as pl
from jax.experimental.pallas import tpu as pltpu


def _copy_body(u_in, i_in, u_out, i_out, sem_u, sem_i):
    cu = pltpu.make_async_copy(u_in, u_out, sem_u)
    ci = pltpu.make_async_copy(i_in, i_out, sem_i)
    cu.start()
    ci.start()
    cu.wait()
    ci.wait()


def kernel(emb_user, emb_item):
    u_flat = emb_user.reshape(-1)
    i_flat = emb_item.reshape(-1)
    u, i = pl.pallas_call(
        _copy_body,
        in_specs=[
            pl.BlockSpec(memory_space=pl.ANY),
            pl.BlockSpec(memory_space=pl.ANY),
        ],
        out_specs=[
            pl.BlockSpec(memory_space=pl.ANY),
            pl.BlockSpec(memory_space=pl.ANY),
        ],
        out_shape=[
            jax.ShapeDtypeStruct(u_flat.shape, u_flat.dtype),
            jax.ShapeDtypeStruct(i_flat.shape, i_flat.dtype),
        ],
        scratch_shapes=[pltpu.SemaphoreType.DMA, pltpu.SemaphoreType.DMA],
    )(u_flat, i_flat)
    return (u.reshape(emb_user.shape), i.reshape(emb_item.shape))


# speedup vs baseline: 3.3165x; 1.0004x over previous
import jax
import jax.numpy as jnp
from jax.experimental import pallas as pl
from jax.experimental.pallas import tpu as pltpu


_NCHUNK_U = 50
_NCHUNK_I = 5


def _copy_body(u_in, i_in, u_out, i_out, sem_u, sem_i):
    cu = u_in.shape[0] // _NCHUNK_U
    ci = i_in.shape[0] // _NCHUNK_I
    copies = []
    for k in range(_NCHUNK_U):
        c = pltpu.make_async_copy(
            u_in.at[pl.ds(k * cu, cu)], u_out.at[pl.ds(k * cu, cu)], sem_u.at[k]
        )
        c.start()
        copies.append(c)
    for k in range(_NCHUNK_I):
        c = pltpu.make_async_copy(
            i_in.at[pl.ds(k * ci, ci)], i_out.at[pl.ds(k * ci, ci)], sem_i.at[k]
        )
        c.start()
        copies.append(c)
    for c in copies:
        c.wait()


def kernel(emb_user, emb_item):
    u_flat = emb_user.reshape(-1)
    i_flat = emb_item.reshape(-1)
    u, i = pl.pallas_call(
        _copy_body,
        in_specs=[
            pl.BlockSpec(memory_space=pl.ANY),
            pl.BlockSpec(memory_space=pl.ANY),
        ],
        out_specs=[
            pl.BlockSpec(memory_space=pl.ANY),
            pl.BlockSpec(memory_space=pl.ANY),
        ],
        out_shape=[
            jax.ShapeDtypeStruct(u_flat.shape, u_flat.dtype),
            jax.ShapeDtypeStruct(i_flat.shape, i_flat.dtype),
        ],
        scratch_shapes=[
            pltpu.SemaphoreType.DMA((_NCHUNK_U,)),
            pltpu.SemaphoreType.DMA((_NCHUNK_I,)),
        ],
    )(u_flat, i_flat)
    return (u.reshape(emb_user.shape), i.reshape(emb_item.shape))


# speedup vs baseline: 14.7398x; 4.4444x over previous
import jax
import jax.numpy as jnp
from jax.experimental import pallas as pl
from jax.experimental.pallas import tpu as pltpu

_GRID = 125
_U_ROWS = 250000 // _GRID
_I_ROWS = 25000 // _GRID


def _copy_body(u_in, i_in, u_out, i_out):
    u_out[...] = u_in[...]
    i_out[...] = i_in[...]


def kernel(emb_user, emb_item):
    u2 = emb_user.reshape(250000, 128)
    i2 = emb_item.reshape(25000, 128)
    u, i = pl.pallas_call(
        _copy_body,
        grid=(_GRID,),
        in_specs=[
            pl.BlockSpec((_U_ROWS, 128), lambda g: (g, 0)),
            pl.BlockSpec((_I_ROWS, 128), lambda g: (g, 0)),
        ],
        out_specs=[
            pl.BlockSpec((_U_ROWS, 128), lambda g: (g, 0)),
            pl.BlockSpec((_I_ROWS, 128), lambda g: (g, 0)),
        ],
        out_shape=[
            jax.ShapeDtypeStruct(u2.shape, u2.dtype),
            jax.ShapeDtypeStruct(i2.shape, i2.dtype),
        ],
        compiler_params=pltpu.CompilerParams(
            dimension_semantics=("parallel",),
        ),
    )(u2, i2)
    return (u.reshape(emb_user.shape), i.reshape(emb_item.shape))


# speedup vs baseline: 15.2775x; 1.0365x over previous
import jax
import jax.numpy as jnp
from jax.experimental import pallas as pl
from jax.experimental.pallas import tpu as pltpu

_GRID = 25
_U_ROWS = 250000 // _GRID
_I_ROWS = 25000 // _GRID


def _copy_body(u_in, i_in, u_out, i_out):
    u_out[...] = u_in[...]
    i_out[...] = i_in[...]


def kernel(emb_user, emb_item):
    u2 = emb_user.reshape(250000, 128)
    i2 = emb_item.reshape(25000, 128)
    u, i = pl.pallas_call(
        _copy_body,
        grid=(_GRID,),
        in_specs=[
            pl.BlockSpec((_U_ROWS, 128), lambda g: (g, 0)),
            pl.BlockSpec((_I_ROWS, 128), lambda g: (g, 0)),
        ],
        out_specs=[
            pl.BlockSpec((_U_ROWS, 128), lambda g: (g, 0)),
            pl.BlockSpec((_I_ROWS, 128), lambda g: (g, 0)),
        ],
        out_shape=[
            jax.ShapeDtypeStruct(u2.shape, u2.dtype),
            jax.ShapeDtypeStruct(i2.shape, i2.dtype),
        ],
        compiler_params=pltpu.CompilerParams(
            dimension_semantics=("parallel",),
        ),
    )(u2, i2)
    return (u.reshape(emb_user.shape), i.reshape(emb_item.shape))
